# baseline (speedup 1.0000x reference)
import jax
import jax.numpy as jnp
from jax.experimental import pallas as pl
from jax.experimental.pallas import tpu as pltpu

_B, _C, _H, _W = 4, 512, 64, 64
_HW = _H * _W


def _rpn_body(x_ref, wt_ref, b3_ref, wh_ref, bh_ref, box_ref, cls_ref):
    lane = jax.lax.broadcasted_iota(jnp.int32, (1, _HW), 1)
    wcol = lane % _W
    zb = jnp.bfloat16(0)

    xc = x_ref[0].astype(jnp.bfloat16)
    xm = jnp.where(lane < _W, zb, pltpu.roll(xc, _W, axis=1))
    xp = jnp.where(lane >= _HW - _W, zb,
                   pltpu.roll(xc, _HW - _W, axis=1))
    xcat = jnp.concatenate([xm, xc, xp], axis=0)

    def z_for(dx):
        return jnp.dot(wt_ref[dx + 1], xcat,
                       preferred_element_type=jnp.float32)

    acc = z_for(0)
    zm = pltpu.roll(z_for(-1), 1, axis=1)
    acc += jnp.where(wcol == 0, 0.0, zm)
    zp = pltpu.roll(z_for(1), _HW - 1, axis=1)
    acc += jnp.where(wcol == _W - 1, 0.0, zp)

    h = jnp.maximum(acc + b3_ref[...], 0.0).astype(jnp.bfloat16)
    o = jnp.dot(wh_ref[...], h, preferred_element_type=jnp.float32)
    ot = jnp.transpose(o) + bh_ref[...]
    box_ref[0] = ot[:, :36]
    cls_ref[0] = ot[:, 36:45]


@jax.jit
def kernel(x, conv3_w, conv3_b, reg_w, reg_b, cls_w, cls_b):
    B = x.shape[0]
    xf = x.reshape(B, _C, _HW)
    w9 = jnp.transpose(conv3_w, (2, 3, 0, 1)).reshape(9, _C, _C)
    wt = jnp.stack([
        jnp.concatenate([w9[kx], w9[kx + 3], w9[kx + 6]], axis=1)
        for kx in range(3)]).astype(jnp.bfloat16)
    wh = jnp.concatenate(
        [reg_w.reshape(36, _C), cls_w.reshape(9, _C),
         jnp.zeros((64 - 45, _C), jnp.float32)], axis=0).astype(jnp.bfloat16)
    bh = jnp.concatenate([reg_b, cls_b, jnp.zeros((64 - 45,), jnp.float32)])

    out = pl.pallas_call(
        _rpn_body,
        grid=(B,),
        in_specs=[
            pl.BlockSpec((1, _C, _HW), lambda b: (b, 0, 0)),
            pl.BlockSpec((3, _C, 3 * _C), lambda b: (0, 0, 0)),
            pl.BlockSpec((_C, 1), lambda b: (0, 0)),
            pl.BlockSpec((64, _C), lambda b: (0, 0)),
            pl.BlockSpec((1, 64), lambda b: (0, 0)),
        ],
        out_specs=[pl.BlockSpec((1, _HW, 36), lambda b: (b, 0, 0)),
                   pl.BlockSpec((1, _HW, 9), lambda b: (b, 0, 0))],
        out_shape=[jax.ShapeDtypeStruct((B, _HW, 36), jnp.float32),
                   jax.ShapeDtypeStruct((B, _HW, 9), jnp.float32)],
    )(xf, wt, conv3_b.reshape(_C, 1), wh, bh.reshape(1, 64))

    box, cls = out
    return (box.reshape(B, _HW * 9, 4), cls.reshape(B, _HW * 9, 1))


# speedup vs baseline: 1.0119x; 1.0119x over previous
import jax
import jax.numpy as jnp
from jax.experimental import pallas as pl
from jax.experimental.pallas import tpu as pltpu

_B, _C, _H, _W = 4, 512, 64, 64
_HW = _H * _W


def _rpn_body(x_ref, wt_ref, b3_ref, wh_ref, bh_ref, box_ref, cls_ref):
    lane = jax.lax.broadcasted_iota(jnp.int32, (1, _HW), 1)
    wcol = lane % _W
    zb = jnp.bfloat16(0)

    xc = x_ref[0].astype(jnp.bfloat16)
    xs = {0: xc}
    xs[-1] = jnp.where(lane < _W, zb, pltpu.roll(xc, _W, axis=1))
    xs[1] = jnp.where(lane >= _HW - _W, zb,
                      pltpu.roll(xc, _HW - _W, axis=1))

    def z_for(dx):
        k0 = dx + 1
        z = jnp.dot(wt_ref[k0], xs[-1], preferred_element_type=jnp.float32)
        z += jnp.dot(wt_ref[k0 + 3], xs[0], preferred_element_type=jnp.float32)
        z += jnp.dot(wt_ref[k0 + 6], xs[1], preferred_element_type=jnp.float32)
        return z

    acc = z_for(0)
    zm = pltpu.roll(z_for(-1), 1, axis=1)
    acc += jnp.where(wcol == 0, 0.0, zm)
    zp = pltpu.roll(z_for(1), _HW - 1, axis=1)
    acc += jnp.where(wcol == _W - 1, 0.0, zp)

    h = jnp.maximum(acc + b3_ref[...], 0.0).astype(jnp.bfloat16)
    o = jnp.dot(wh_ref[...], h, preferred_element_type=jnp.float32)
    ot = jnp.transpose(o) + bh_ref[...]
    box_ref[0] = ot[:, :36]
    cls_ref[0] = ot[:, 36:45]


@jax.jit
def kernel(x, conv3_w, conv3_b, reg_w, reg_b, cls_w, cls_b):
    B = x.shape[0]
    xf = x.reshape(B, _C, _HW)
    wt = jnp.transpose(conv3_w, (2, 3, 0, 1)).reshape(9, _C, _C)
    wt = wt.astype(jnp.bfloat16)
    wh = jnp.concatenate(
        [reg_w.reshape(36, _C), cls_w.reshape(9, _C),
         jnp.zeros((64 - 45, _C), jnp.float32)], axis=0).astype(jnp.bfloat16)
    bh = jnp.concatenate([reg_b, cls_b, jnp.zeros((64 - 45,), jnp.float32)])

    out = pl.pallas_call(
        _rpn_body,
        grid=(B,),
        in_specs=[
            pl.BlockSpec((1, _C, _HW), lambda b: (b, 0, 0)),
            pl.BlockSpec((9, _C, _C), lambda b: (0, 0, 0)),
            pl.BlockSpec((_C, 1), lambda b: (0, 0)),
            pl.BlockSpec((64, _C), lambda b: (0, 0)),
            pl.BlockSpec((1, 64), lambda b: (0, 0)),
        ],
        out_specs=[pl.BlockSpec((1, _HW, 36), lambda b: (b, 0, 0)),
                   pl.BlockSpec((1, _HW, 9), lambda b: (b, 0, 0))],
        out_shape=[jax.ShapeDtypeStruct((B, _HW, 36), jnp.float32),
                   jax.ShapeDtypeStruct((B, _HW, 9), jnp.float32)],
    )(xf, wt, conv3_b.reshape(_C, 1), wh, bh.reshape(1, 64))

    box, cls = out
    return (box.reshape(B, _HW * 9, 4), cls.reshape(B, _HW * 9, 1))
